# baseline (speedup 1.0000x reference)
import functools

import jax
import jax.numpy as jnp
from jax.experimental import pallas as pl
from jax.experimental.pallas import tpu as pltpu


def _fused_body(x_ref, adj_ref, w1_ref, b1_ref, w2_ref, b2_ref, out_ref, *, tm):
    i = pl.program_id(0)
    agg = jnp.dot(adj_ref[...], x_ref[...], preferred_element_type=jnp.float32)
    h = agg + x_ref[pl.ds(i * tm, tm), :]
    h = jnp.maximum(jnp.dot(h, w1_ref[...], preferred_element_type=jnp.float32)
                    + b1_ref[...], 0.0)
    out_ref[...] = (jnp.dot(h, w2_ref[...], preferred_element_type=jnp.float32)
                    + b2_ref[...])


@functools.partial(jax.jit, static_argnames=())
def _run(x2, adj, W1, b1r, W2, b2r):
    n, d = x2.shape
    tm = 200
    grid = (n // tm,)
    return pl.pallas_call(
        functools.partial(_fused_body, tm=tm),
        grid=grid,
        in_specs=[
            pl.BlockSpec((n, d), lambda i: (0, 0)),
            pl.BlockSpec((tm, n), lambda i: (i, 0)),
            pl.BlockSpec((d, d), lambda i: (0, 0)),
            pl.BlockSpec((1, d), lambda i: (0, 0)),
            pl.BlockSpec((d, d), lambda i: (0, 0)),
            pl.BlockSpec((1, d), lambda i: (0, 0)),
        ],
        out_specs=pl.BlockSpec((tm, d), lambda i: (i, 0)),
        out_shape=jax.ShapeDtypeStruct((n, d), jnp.float32),
        compiler_params=pltpu.CompilerParams(
            dimension_semantics=("arbitrary",),
        ),
    )(x2, adj, W1, b1r, W2, b2r)


def kernel(x, adj, W1, b1, W2, b2):
    if adj.ndim == 3:
        adj = adj[0]
    x2 = x[0]
    out = _run(x2, adj, W1, b1.reshape(1, -1), W2, b2.reshape(1, -1))
    return out[None]


# speedup vs baseline: 1.0272x; 1.0272x over previous
import functools

import jax
import jax.numpy as jnp
from jax import lax
from jax.experimental import pallas as pl
from jax.experimental.pallas import tpu as pltpu

_TM = 200
_NBUF = 4


def _body(x_ref, adj_hbm, w1_ref, b1_ref, w2_ref, b2_ref, out_ref,
          buf, sem, *, tm, nbuf):
    n = x_ref.shape[0]
    num_panels = n // tm

    def panel_copy(i, slot):
        return pltpu.make_async_copy(
            adj_hbm.at[pl.ds(i * tm, tm), :], buf.at[slot], sem.at[slot])

    for s in range(nbuf):
        panel_copy(s, s).start()

    def step(i, carry):
        slot = lax.rem(i, nbuf)
        panel_copy(i, slot).wait()
        agg = jnp.dot(buf[slot], x_ref[...],
                      preferred_element_type=jnp.float32)

        @pl.when(i + nbuf < num_panels)
        def _():
            panel_copy(i + nbuf, slot).start()

        h = agg + x_ref[pl.ds(i * tm, tm), :]
        h = jnp.maximum(
            jnp.dot(h, w1_ref[...], preferred_element_type=jnp.float32)
            + b1_ref[...], 0.0)
        out_ref[pl.ds(i * tm, tm), :] = (
            jnp.dot(h, w2_ref[...], preferred_element_type=jnp.float32)
            + b2_ref[...])
        return carry

    lax.fori_loop(0, num_panels, step, 0)


@jax.jit
def _run(x2, adj, W1, b1r, W2, b2r):
    n, d = x2.shape
    return pl.pallas_call(
        functools.partial(_body, tm=_TM, nbuf=_NBUF),
        in_specs=[
            pl.BlockSpec(memory_space=pltpu.VMEM),
            pl.BlockSpec(memory_space=pl.ANY),
            pl.BlockSpec(memory_space=pltpu.VMEM),
            pl.BlockSpec(memory_space=pltpu.VMEM),
            pl.BlockSpec(memory_space=pltpu.VMEM),
            pl.BlockSpec(memory_space=pltpu.VMEM),
        ],
        out_specs=pl.BlockSpec(memory_space=pltpu.VMEM),
        out_shape=jax.ShapeDtypeStruct((n, d), jnp.float32),
        scratch_shapes=[
            pltpu.VMEM((_NBUF, _TM, n), jnp.float32),
            pltpu.SemaphoreType.DMA((_NBUF,)),
        ],
    )(x2, adj, W1, b1r, W2, b2r)


def kernel(x, adj, W1, b1, W2, b2):
    if adj.ndim == 3:
        adj = adj[0]
    x2 = x[0]
    out = _run(x2, adj, W1, b1.reshape(1, -1), W2, b2.reshape(1, -1))
    return out[None]


# speedup vs baseline: 1.0413x; 1.0138x over previous
import functools

import jax
import jax.numpy as jnp
from jax.experimental import pallas as pl
from jax.experimental.pallas import tpu as pltpu


def _fused_body(x_ref, adj_ref, w1_ref, b1_ref, w2_ref, b2_ref, out_ref, *, tm):
    i = pl.program_id(0)
    agg = jnp.dot(adj_ref[...], x_ref[...], preferred_element_type=jnp.float32)
    h = agg + x_ref[pl.ds(i * tm, tm), :]
    h = jnp.maximum(jnp.dot(h, w1_ref[...], preferred_element_type=jnp.float32)
                    + b1_ref[...], 0.0)
    out_ref[...] = (jnp.dot(h, w2_ref[...], preferred_element_type=jnp.float32)
                    + b2_ref[...])


@jax.jit
def _run(x2, adj, W1, b1r, W2, b2r):
    n, d = x2.shape
    tm = 400
    grid = (n // tm,)
    return pl.pallas_call(
        functools.partial(_fused_body, tm=tm),
        grid=grid,
        in_specs=[
            pl.BlockSpec((n, d), lambda i: (0, 0)),
            pl.BlockSpec((tm, n), lambda i: (i, 0)),
            pl.BlockSpec((d, d), lambda i: (0, 0)),
            pl.BlockSpec((1, d), lambda i: (0, 0)),
            pl.BlockSpec((d, d), lambda i: (0, 0)),
            pl.BlockSpec((1, d), lambda i: (0, 0)),
        ],
        out_specs=pl.BlockSpec((tm, d), lambda i: (i, 0)),
        out_shape=jax.ShapeDtypeStruct((n, d), jnp.float32),
        compiler_params=pltpu.CompilerParams(
            dimension_semantics=("arbitrary",),
        ),
    )(x2, adj, W1, b1r, W2, b2r)


def kernel(x, adj, W1, b1, W2, b2):
    if adj.ndim == 3:
        adj = adj[0]
    x2 = x[0]
    out = _run(x2, adj, W1, b1.reshape(1, -1), W2, b2.reshape(1, -1))
    return out[None]
